# Initial kernel scaffold; baseline (speedup 1.0000x reference)
import jax
import jax.numpy as jnp
from jax.experimental import pallas as pl


def kernel(x, edge_index, We1, be1, g1, bt1, We2, be2, g2, bt2, We3, be3, Wc1, bc1, gn1, bnb1, Wc2, bc2, gn2, bnb2, Wc3, bc3, gn3, bnb3, Wh, bh):
    raise NotImplementedError("write your pallas kernel here")


# speedup vs baseline: 7.3296x; 7.3296x over previous
import functools

import jax
import jax.numpy as jnp
from jax import lax
from jax.experimental import pallas as pl
from jax.experimental.pallas import tpu as pltpu
from jax.experimental.pallas import tpu_sc as plsc

N = 10000
E = 320000
HID = 256
HALF = 128
NLAB = 40
NCORE = 2
NSUB = 16
NPAD = 10240
CHUNK = 128
CPS = 160
SUP = 8
NSUP = CPS // SUP
EPW = CHUNK * CPS
EPAD = EPW * NSUB
RPS = NPAD // NSUB
BN_EPS = 1e-5

_f32 = jnp.float32


def _sc_mesh():
    return plsc.VectorSubcoreMesh(core_axis_name="c", subcore_axis_name="s")


def _sc_degree(dstw):
    hcps = CPS // 2

    @functools.partial(
        pl.kernel,
        out_type=jax.ShapeDtypeStruct((NCORE, NPAD), _f32),
        mesh=_sc_mesh(),
        scratch_types=[
            pltpu.VMEM((hcps, CHUNK), jnp.int32),
            pltpu.VMEM((CHUNK,), _f32),
            pltpu.VMEM((RPS,), _f32),
            pltpu.VMEM_SHARED((NPAD,), _f32),
            pltpu.SemaphoreType.DMA,
        ],
    )
    def deg_kernel(dst_hbm, out_hbm, idx_v, ones_v, zero_v, acc_sh, sem):
        c = lax.axis_index("c")
        s = lax.axis_index("s")
        wid = s * NCORE + c

        @pl.loop(0, CHUNK, step=16)
        def _(i):
            ones_v[pl.ds(i, 16)] = jnp.full((16,), 1.0, _f32)

        @pl.loop(0, RPS, step=16)
        def _(i):
            zero_v[pl.ds(i, 16)] = jnp.zeros((16,), _f32)

        pltpu.sync_copy(zero_v, acc_sh.at[pl.ds(s * RPS, RPS)])
        pltpu.async_copy(dst_hbm.at[wid], idx_v, sem).wait()
        plsc.subcore_barrier()

        @pl.loop(0, hcps)
        def _(j):
            pltpu.sync_copy(ones_v, acc_sh.at[idx_v.at[j]], add=True)

        plsc.subcore_barrier()
        pltpu.sync_copy(acc_sh.at[pl.ds(s * RPS, RPS)],
                        out_hbm.at[c, pl.ds(s * RPS, RPS)])

    return deg_kernel(dstw)


def _sc_aggregate(hws, src3, dst3):
    @functools.partial(
        pl.kernel,
        out_type=jax.ShapeDtypeStruct((NCORE, NPAD, HALF), _f32),
        mesh=_sc_mesh(),
        scratch_types=[
            pltpu.VMEM((SUP, CHUNK), jnp.int32),
            pltpu.VMEM((SUP, CHUNK), jnp.int32),
            pltpu.VMEM((SUP, CHUNK), jnp.int32),
            pltpu.VMEM((SUP, CHUNK), jnp.int32),
            pltpu.VMEM((CHUNK, HALF), _f32),
            pltpu.VMEM((CHUNK, HALF), _f32),
            pltpu.VMEM_SHARED((NPAD, HALF), _f32),
            pltpu.SemaphoreType.DMA,
            pltpu.SemaphoreType.DMA,
            pltpu.SemaphoreType.DMA,
            pltpu.SemaphoreType.DMA,
        ],
    )
    def agg_kernel(hws_hbm, src_hbm, dst_hbm, out_hbm,
                   src_a, src_b, dst_a, dst_b, gb_a, gb_b, acc_sh,
                   sem_a, sem_b, sem_ia, sem_ib):
        c = lax.axis_index("c")
        s = lax.axis_index("s")

        pltpu.async_copy(src_hbm.at[s, pl.ds(0, SUP)], src_a, sem_ia)
        pltpu.async_copy(dst_hbm.at[s, pl.ds(0, SUP)], dst_a, sem_ia)
        pltpu.async_copy(hws_hbm.at[c, pl.ds(s * RPS, RPS)],
                         acc_sh.at[pl.ds(s * RPS, RPS)], sem_b).wait()
        plsc.subcore_barrier()

        table = hws_hbm.at[c]

        def _process(srcs, dsts):
            @pl.loop(0, SUP, step=2)
            def _(k):
                cp_a = pltpu.async_copy(table.at[srcs.at[k]], gb_a, sem_a)
                cp_b = pltpu.async_copy(table.at[srcs.at[k + 1]], gb_b, sem_b)
                cp_a.wait()
                pltpu.sync_copy(gb_a, acc_sh.at[dsts.at[k]], add=True)
                cp_b.wait()
                pltpu.sync_copy(gb_b, acc_sh.at[dsts.at[k + 1]], add=True)

        @pl.loop(0, NSUP, step=2)
        def _(u):
            pltpu.make_async_copy(src_hbm.at[s, pl.ds(u * SUP, SUP)],
                                  src_a, sem_ia).wait()
            pltpu.make_async_copy(dst_hbm.at[s, pl.ds(u * SUP, SUP)],
                                  dst_a, sem_ia).wait()
            pltpu.async_copy(src_hbm.at[s, pl.ds((u + 1) * SUP, SUP)],
                             src_b, sem_ib)
            pltpu.async_copy(dst_hbm.at[s, pl.ds((u + 1) * SUP, SUP)],
                             dst_b, sem_ib)
            _process(src_a, dst_a)
            pltpu.make_async_copy(src_hbm.at[s, pl.ds((u + 1) * SUP, SUP)],
                                  src_b, sem_ib).wait()
            pltpu.make_async_copy(dst_hbm.at[s, pl.ds((u + 1) * SUP, SUP)],
                                  dst_b, sem_ib).wait()

            @pl.when(u + 2 < NSUP)
            def _():
                pltpu.async_copy(src_hbm.at[s, pl.ds((u + 2) * SUP, SUP)],
                                 src_a, sem_ia)
                pltpu.async_copy(dst_hbm.at[s, pl.ds((u + 2) * SUP, SUP)],
                                 dst_a, sem_ia)

            _process(src_b, dst_b)

        plsc.subcore_barrier()
        pltpu.sync_copy(acc_sh.at[pl.ds(s * RPS, RPS)],
                        out_hbm.at[c, pl.ds(s * RPS, RPS)])

    return agg_kernel(hws, src3, dst3)


def _bn(t, g, b):
    mu = jnp.mean(t, axis=0)
    var = jnp.mean((t - mu) ** 2, axis=0)
    return g * (t - mu) * lax.rsqrt(var + BN_EPS) + b


def _dis_col(dcol):
    return lax.rsqrt(dcol[0] + dcol[1] + 1.0)[:N]


def _dot(a, b):
    return jnp.dot(a, b, preferred_element_type=_f32)


def _enc_body(x_ref, we1, be1, g1, bt1, we2, be2, g2, bt2, we3, be3, wc1,
              u1_ref):
    h = _bn(_dot(x_ref[...], we1[...]) + be1[...], g1[...], bt1[...])
    h = jnp.maximum(h, 0.0)
    h = _bn(_dot(h, we2[...]) + be2[...], g2[...], bt2[...])
    h = jnp.maximum(h, 0.0)
    lat = _dot(h, we3[...]) + be3[...]
    u1_ref[...] = _dot(lat, wc1[...])


def _write_hws(hws_ref, us):
    hws_ref[0, :N, :] = us[:, :HALF]
    hws_ref[1, :N, :] = us[:, HALF:]
    z = jnp.zeros((NPAD - N, HALF), _f32)
    hws_ref[0, N:, :] = z
    hws_ref[1, N:, :] = z


def _scale_body(u_ref, dcol_ref, hws_ref):
    dis = _dis_col(dcol_ref[...])
    _write_hws(hws_ref, u_ref[...] * dis)


def _post_nores_body(agg_ref, dcol_ref, b_ref, g_ref, bb_ref, h_ref):
    dis = _dis_col(dcol_ref[...])
    t = jnp.concatenate([agg_ref[0, :N, :] * dis, agg_ref[1, :N, :] * dis],
                        axis=1) + b_ref[...]
    h_ref[...] = jnp.maximum(_bn(t, g_ref[...], bb_ref[...]), 0.0)


def _post_res_body(agg_ref, dcol_ref, b_ref, g_ref, bb_ref, res_ref, h_ref):
    dis = _dis_col(dcol_ref[...])
    t = jnp.concatenate([agg_ref[0, :N, :] * dis, agg_ref[1, :N, :] * dis],
                        axis=1) + b_ref[...]
    h_ref[...] = jnp.maximum(_bn(t, g_ref[...], bb_ref[...]) + res_ref[...],
                             0.0)


def _matscale_body(h_ref, dcol_ref, w_ref, hws_ref):
    dis = _dis_col(dcol_ref[...])
    _write_hws(hws_ref, _dot(h_ref[...], w_ref[...]) * dis)


def _head_body(agg_ref, dcol_ref, b_ref, g_ref, bb_ref, res_ref, wh_ref,
               bh_ref, out_ref):
    dis = _dis_col(dcol_ref[...])
    t = jnp.concatenate([agg_ref[0, :N, :] * dis, agg_ref[1, :N, :] * dis],
                        axis=1) + b_ref[...]
    h3 = jnp.maximum(_bn(t, g_ref[...], bb_ref[...]) + res_ref[...], 0.0)
    out_ref[...] = _dot(h3, wh_ref[...]) + bh_ref[...]


def _tc(body, out_shape, *args):
    return pl.pallas_call(
        body, out_shape=jax.ShapeDtypeStruct(out_shape, _f32))(*args)


def kernel(x, edge_index, We1, be1, g1, bt1, We2, be2, g2, bt2, We3, be3,
           Wc1, bc1, gn1, bnb1, Wc2, bc2, gn2, bnb2, Wc3, bc3, gn3, bnb3,
           Wh, bh):
    npad_e = EPAD - E
    src_p = jnp.concatenate(
        [edge_index[0], jnp.zeros((npad_e,), jnp.int32)]).reshape(
            NSUB, CPS, CHUNK)
    dst_p = jnp.concatenate(
        [edge_index[1],
         N + (jnp.arange(npad_e, dtype=jnp.int32) % (NPAD - N))]).reshape(
            NSUB, CPS, CHUNK)

    dst_w = dst_p.reshape(NCORE * NSUB, CPS // 2, CHUNK)
    deg2 = _sc_degree(dst_w)
    dcol = deg2.reshape(NCORE, NPAD, 1)

    u1 = _tc(_enc_body, (N, HID), x, We1, be1, g1, bt1, We2, be2, g2, bt2,
             We3, be3, Wc1)

    hws1 = _tc(_scale_body, (NCORE, NPAD, HALF), u1, dcol)
    agg1 = _sc_aggregate(hws1, src_p, dst_p)
    h1 = _tc(_post_nores_body, (N, HID), agg1, dcol, bc1, gn1, bnb1)

    hws2 = _tc(_matscale_body, (NCORE, NPAD, HALF), h1, dcol, Wc2)
    agg2 = _sc_aggregate(hws2, src_p, dst_p)
    h2 = _tc(_post_res_body, (N, HID), agg2, dcol, bc2, gn2, bnb2, h1)

    hws3 = _tc(_matscale_body, (NCORE, NPAD, HALF), h2, dcol, Wc3)
    agg3 = _sc_aggregate(hws3, src_p, dst_p)
    logits = _tc(_head_body, (N, NLAB), agg3, dcol, bc3, gn3, bnb3, h2,
                 Wh, bh)
    return logits


# speedup vs baseline: 8.1305x; 1.1093x over previous
import functools

import jax
import jax.numpy as jnp
from jax import lax
from jax.experimental import pallas as pl
from jax.experimental.pallas import tpu as pltpu
from jax.experimental.pallas import tpu_sc as plsc

N = 10000
E = 320000
HID = 256
HALF = 128
NLAB = 40
NCORE = 2
NSUB = 16
NPAD = 10240
CHUNK = 128
CPS = 160
SUP = 8
NSUP = CPS // SUP
EPW = CHUNK * CPS
EPAD = EPW * NSUB
RPS = NPAD // NSUB
BN_EPS = 1e-5

_f32 = jnp.float32


def _sc_mesh():
    return plsc.VectorSubcoreMesh(core_axis_name="c", subcore_axis_name="s")


def _sc_degree(dstw):
    hcps = CPS // 2

    @functools.partial(
        pl.kernel,
        out_type=jax.ShapeDtypeStruct((NCORE, NPAD), _f32),
        mesh=_sc_mesh(),
        scratch_types=[
            pltpu.VMEM((hcps, CHUNK), jnp.int32),
            pltpu.VMEM((CHUNK,), _f32),
            pltpu.VMEM((RPS,), _f32),
            pltpu.VMEM_SHARED((NPAD,), _f32),
            pltpu.SemaphoreType.DMA,
        ],
    )
    def deg_kernel(dst_hbm, out_hbm, idx_v, ones_v, zero_v, acc_sh, sem):
        c = lax.axis_index("c")
        s = lax.axis_index("s")
        wid = s * NCORE + c

        @pl.loop(0, CHUNK, step=16)
        def _(i):
            ones_v[pl.ds(i, 16)] = jnp.full((16,), 1.0, _f32)

        @pl.loop(0, RPS, step=16)
        def _(i):
            zero_v[pl.ds(i, 16)] = jnp.zeros((16,), _f32)

        pltpu.sync_copy(zero_v, acc_sh.at[pl.ds(s * RPS, RPS)])
        pltpu.async_copy(dst_hbm.at[wid], idx_v, sem).wait()
        plsc.subcore_barrier()

        @pl.loop(0, hcps)
        def _(j):
            pltpu.sync_copy(ones_v, acc_sh.at[idx_v.at[j]], add=True)

        plsc.subcore_barrier()
        pltpu.sync_copy(acc_sh.at[pl.ds(s * RPS, RPS)],
                        out_hbm.at[c, pl.ds(s * RPS, RPS)])

    return deg_kernel(dstw)


def _sc_aggregate(hws, src3, dst3):
    @functools.partial(
        pl.kernel,
        out_type=jax.ShapeDtypeStruct((NCORE, NPAD, HALF), _f32),
        mesh=_sc_mesh(),
        scratch_types=[
            pltpu.VMEM((SUP, CHUNK), jnp.int32),
            pltpu.VMEM((SUP, CHUNK), jnp.int32),
            pltpu.VMEM((SUP, CHUNK), jnp.int32),
            pltpu.VMEM((SUP, CHUNK), jnp.int32),
            pltpu.VMEM((CHUNK, HALF), _f32),
            pltpu.VMEM((CHUNK, HALF), _f32),
            pltpu.VMEM_SHARED((NPAD, HALF), _f32),
            pltpu.SemaphoreType.DMA,
            pltpu.SemaphoreType.DMA,
            pltpu.SemaphoreType.DMA,
            pltpu.SemaphoreType.DMA,
        ],
    )
    def agg_kernel(hws_hbm, src_hbm, dst_hbm, out_hbm,
                   src_a, src_b, dst_a, dst_b, gb_a, gb_b, acc_sh,
                   sem_a, sem_b, sem_ia, sem_ib):
        c = lax.axis_index("c")
        s = lax.axis_index("s")

        pltpu.async_copy(src_hbm.at[s, pl.ds(0, SUP)], src_a, sem_ia)
        pltpu.async_copy(dst_hbm.at[s, pl.ds(0, SUP)], dst_a, sem_ia)
        pltpu.async_copy(hws_hbm.at[c, pl.ds(s * RPS, RPS)],
                         acc_sh.at[pl.ds(s * RPS, RPS)], sem_b).wait()
        plsc.subcore_barrier()

        table = hws_hbm.at[c]

        def _process(srcs, dsts):
            pltpu.async_copy(table.at[srcs.at[0]], gb_a, sem_a)
            pltpu.async_copy(table.at[srcs.at[1]], gb_b, sem_b)

            @pl.loop(0, SUP, step=2)
            def _(k):
                pltpu.make_async_copy(table.at[srcs.at[k]], gb_a, sem_a).wait()
                pltpu.sync_copy(gb_a, acc_sh.at[dsts.at[k]], add=True)

                @pl.when(k + 2 < SUP)
                def _():
                    pltpu.async_copy(table.at[srcs.at[k + 2]], gb_a, sem_a)

                pltpu.make_async_copy(table.at[srcs.at[k + 1]], gb_b,
                                      sem_b).wait()
                pltpu.sync_copy(gb_b, acc_sh.at[dsts.at[k + 1]], add=True)

                @pl.when(k + 3 < SUP)
                def _():
                    pltpu.async_copy(table.at[srcs.at[k + 3]], gb_b, sem_b)

        @pl.loop(0, NSUP, step=2)
        def _(u):
            pltpu.make_async_copy(src_hbm.at[s, pl.ds(u * SUP, SUP)],
                                  src_a, sem_ia).wait()
            pltpu.make_async_copy(dst_hbm.at[s, pl.ds(u * SUP, SUP)],
                                  dst_a, sem_ia).wait()
            pltpu.async_copy(src_hbm.at[s, pl.ds((u + 1) * SUP, SUP)],
                             src_b, sem_ib)
            pltpu.async_copy(dst_hbm.at[s, pl.ds((u + 1) * SUP, SUP)],
                             dst_b, sem_ib)
            _process(src_a, dst_a)
            pltpu.make_async_copy(src_hbm.at[s, pl.ds((u + 1) * SUP, SUP)],
                                  src_b, sem_ib).wait()
            pltpu.make_async_copy(dst_hbm.at[s, pl.ds((u + 1) * SUP, SUP)],
                                  dst_b, sem_ib).wait()

            @pl.when(u + 2 < NSUP)
            def _():
                pltpu.async_copy(src_hbm.at[s, pl.ds((u + 2) * SUP, SUP)],
                                 src_a, sem_ia)
                pltpu.async_copy(dst_hbm.at[s, pl.ds((u + 2) * SUP, SUP)],
                                 dst_a, sem_ia)

            _process(src_b, dst_b)

        plsc.subcore_barrier()
        pltpu.sync_copy(acc_sh.at[pl.ds(s * RPS, RPS)],
                        out_hbm.at[c, pl.ds(s * RPS, RPS)])

    return agg_kernel(hws, src3, dst3)


def _bn(t, g, b):
    mu = jnp.mean(t, axis=0)
    var = jnp.mean((t - mu) ** 2, axis=0)
    return g * (t - mu) * lax.rsqrt(var + BN_EPS) + b


def _dis_col(dcol):
    return lax.rsqrt(dcol[0] + dcol[1] + 1.0)[:N]


def _dot(a, b):
    return jnp.dot(a, b, preferred_element_type=_f32)


def _enc_body(x_ref, we1, be1, g1, bt1, we2, be2, g2, bt2, we3, be3, wc1,
              u1_ref):
    h = _bn(_dot(x_ref[...], we1[...]) + be1[...], g1[...], bt1[...])
    h = jnp.maximum(h, 0.0)
    h = _bn(_dot(h, we2[...]) + be2[...], g2[...], bt2[...])
    h = jnp.maximum(h, 0.0)
    lat = _dot(h, we3[...]) + be3[...]
    u1_ref[...] = _dot(lat, wc1[...])


def _write_hws(hws_ref, us):
    hws_ref[0, :N, :] = us[:, :HALF]
    hws_ref[1, :N, :] = us[:, HALF:]
    z = jnp.zeros((NPAD - N, HALF), _f32)
    hws_ref[0, N:, :] = z
    hws_ref[1, N:, :] = z


def _scale_body(u_ref, dcol_ref, hws_ref):
    dis = _dis_col(dcol_ref[...])
    _write_hws(hws_ref, u_ref[...] * dis)


def _post_nores_body(agg_ref, dcol_ref, b_ref, g_ref, bb_ref, h_ref):
    dis = _dis_col(dcol_ref[...])
    t = jnp.concatenate([agg_ref[0, :N, :] * dis, agg_ref[1, :N, :] * dis],
                        axis=1) + b_ref[...]
    h_ref[...] = jnp.maximum(_bn(t, g_ref[...], bb_ref[...]), 0.0)


def _post_res_body(agg_ref, dcol_ref, b_ref, g_ref, bb_ref, res_ref, h_ref):
    dis = _dis_col(dcol_ref[...])
    t = jnp.concatenate([agg_ref[0, :N, :] * dis, agg_ref[1, :N, :] * dis],
                        axis=1) + b_ref[...]
    h_ref[...] = jnp.maximum(_bn(t, g_ref[...], bb_ref[...]) + res_ref[...],
                             0.0)


def _matscale_body(h_ref, dcol_ref, w_ref, hws_ref):
    dis = _dis_col(dcol_ref[...])
    _write_hws(hws_ref, _dot(h_ref[...], w_ref[...]) * dis)


def _head_body(agg_ref, dcol_ref, b_ref, g_ref, bb_ref, res_ref, wh_ref,
               bh_ref, out_ref):
    dis = _dis_col(dcol_ref[...])
    t = jnp.concatenate([agg_ref[0, :N, :] * dis, agg_ref[1, :N, :] * dis],
                        axis=1) + b_ref[...]
    h3 = jnp.maximum(_bn(t, g_ref[...], bb_ref[...]) + res_ref[...], 0.0)
    out_ref[...] = _dot(h3, wh_ref[...]) + bh_ref[...]


def _tc(body, out_shape, *args):
    return pl.pallas_call(
        body, out_shape=jax.ShapeDtypeStruct(out_shape, _f32))(*args)


def kernel(x, edge_index, We1, be1, g1, bt1, We2, be2, g2, bt2, We3, be3,
           Wc1, bc1, gn1, bnb1, Wc2, bc2, gn2, bnb2, Wc3, bc3, gn3, bnb3,
           Wh, bh):
    npad_e = EPAD - E
    src_p = jnp.concatenate(
        [edge_index[0], jnp.zeros((npad_e,), jnp.int32)]).reshape(
            NSUB, CPS, CHUNK)
    dst_p = jnp.concatenate(
        [edge_index[1],
         N + (jnp.arange(npad_e, dtype=jnp.int32) % (NPAD - N))]).reshape(
            NSUB, CPS, CHUNK)

    dst_w = dst_p.reshape(NCORE * NSUB, CPS // 2, CHUNK)
    deg2 = _sc_degree(dst_w)
    dcol = deg2.reshape(NCORE, NPAD, 1)

    u1 = _tc(_enc_body, (N, HID), x, We1, be1, g1, bt1, We2, be2, g2, bt2,
             We3, be3, Wc1)

    hws1 = _tc(_scale_body, (NCORE, NPAD, HALF), u1, dcol)
    agg1 = _sc_aggregate(hws1, src_p, dst_p)
    h1 = _tc(_post_nores_body, (N, HID), agg1, dcol, bc1, gn1, bnb1)

    hws2 = _tc(_matscale_body, (NCORE, NPAD, HALF), h1, dcol, Wc2)
    agg2 = _sc_aggregate(hws2, src_p, dst_p)
    h2 = _tc(_post_res_body, (N, HID), agg2, dcol, bc2, gn2, bnb2, h1)

    hws3 = _tc(_matscale_body, (NCORE, NPAD, HALF), h2, dcol, Wc3)
    agg3 = _sc_aggregate(hws3, src_p, dst_p)
    logits = _tc(_head_body, (N, NLAB), agg3, dcol, bc3, gn3, bnb3, h2,
                 Wh, bh)
    return logits


# speedup vs baseline: 8.2040x; 1.0090x over previous
import functools

import jax
import jax.numpy as jnp
from jax import lax
from jax.experimental import pallas as pl
from jax.experimental.pallas import tpu as pltpu
from jax.experimental.pallas import tpu_sc as plsc

N = 10000
E = 320000
HID = 256
HALF = 128
NLAB = 40
NCORE = 2
NSUB = 16
NPAD = 10240
CHUNK = 64
CPS = 320
SUP = 32
NSUP = CPS // SUP
EPW = CHUNK * CPS
EPAD = EPW * NSUB
RPS = NPAD // NSUB
BN_EPS = 1e-5

_f32 = jnp.float32


def _sc_mesh():
    return plsc.VectorSubcoreMesh(core_axis_name="c", subcore_axis_name="s")


def _sc_degree(dstw):
    hcps = CPS // 2

    @functools.partial(
        pl.kernel,
        out_type=jax.ShapeDtypeStruct((NCORE, NPAD), _f32),
        mesh=_sc_mesh(),
        scratch_types=[
            pltpu.VMEM((hcps, CHUNK), jnp.int32),
            pltpu.VMEM((CHUNK,), _f32),
            pltpu.VMEM((RPS,), _f32),
            pltpu.VMEM_SHARED((NPAD,), _f32),
            pltpu.SemaphoreType.DMA,
        ],
    )
    def deg_kernel(dst_hbm, out_hbm, idx_v, ones_v, zero_v, acc_sh, sem):
        c = lax.axis_index("c")
        s = lax.axis_index("s")
        wid = s * NCORE + c

        @pl.loop(0, CHUNK, step=16)
        def _(i):
            ones_v[pl.ds(i, 16)] = jnp.full((16,), 1.0, _f32)

        @pl.loop(0, RPS, step=16)
        def _(i):
            zero_v[pl.ds(i, 16)] = jnp.zeros((16,), _f32)

        pltpu.sync_copy(zero_v, acc_sh.at[pl.ds(s * RPS, RPS)])
        pltpu.async_copy(dst_hbm.at[wid], idx_v, sem).wait()
        plsc.subcore_barrier()

        @pl.loop(0, hcps)
        def _(j):
            pltpu.sync_copy(ones_v, acc_sh.at[idx_v.at[j]], add=True)

        plsc.subcore_barrier()
        pltpu.sync_copy(acc_sh.at[pl.ds(s * RPS, RPS)],
                        out_hbm.at[c, pl.ds(s * RPS, RPS)])

    return deg_kernel(dstw)


def _sc_aggregate(hws, src3, dst3):
    @functools.partial(
        pl.kernel,
        out_type=jax.ShapeDtypeStruct((NCORE, NPAD, HALF), _f32),
        mesh=_sc_mesh(),
        scratch_types=[
            pltpu.VMEM((SUP, CHUNK), jnp.int32),
            pltpu.VMEM((SUP, CHUNK), jnp.int32),
            pltpu.VMEM((SUP, CHUNK), jnp.int32),
            pltpu.VMEM((SUP, CHUNK), jnp.int32),
            pltpu.VMEM((CHUNK, HALF), _f32),
            pltpu.VMEM((CHUNK, HALF), _f32),
            pltpu.VMEM((CHUNK, HALF), _f32),
            pltpu.VMEM((CHUNK, HALF), _f32),
            pltpu.VMEM_SHARED((NPAD, HALF), _f32),
            pltpu.SemaphoreType.DMA,
            pltpu.SemaphoreType.DMA,
            pltpu.SemaphoreType.DMA,
            pltpu.SemaphoreType.DMA,
            pltpu.SemaphoreType.DMA,
            pltpu.SemaphoreType.DMA,
            pltpu.SemaphoreType.DMA,
            pltpu.SemaphoreType.DMA,
            pltpu.SemaphoreType.DMA,
            pltpu.SemaphoreType.DMA,
        ],
    )
    def agg_kernel(hws_hbm, src_hbm, dst_hbm, out_hbm,
                   src_a, src_b, dst_a, dst_b, gb0, gb1, gb2, gb3, acc_sh,
                   sg0, sg1, sg2, sg3, ss0, ss1, ss2, ss3, sem_ia, sem_ib):
        c = lax.axis_index("c")
        s = lax.axis_index("s")

        pltpu.async_copy(src_hbm.at[s, pl.ds(0, SUP)], src_a, sem_ia)
        pltpu.async_copy(dst_hbm.at[s, pl.ds(0, SUP)], dst_a, sem_ia)
        pltpu.async_copy(hws_hbm.at[c, pl.ds(s * RPS, RPS)],
                         acc_sh.at[pl.ds(s * RPS, RPS)], sg0).wait()
        plsc.subcore_barrier()

        table = hws_hbm.at[c]
        bufs = ((gb0, sg0, ss0), (gb1, sg1, ss1),
                (gb2, sg2, ss2), (gb3, sg3, ss3))

        def _process(srcs, dsts):
            for j, (gb, sg, _) in enumerate(bufs):
                pltpu.async_copy(table.at[srcs.at[j]], gb, sg)

            @pl.loop(0, SUP, step=4)
            def _(k):
                scatters = []
                for j, (gb, sg, ss) in enumerate(bufs):
                    pltpu.make_async_copy(table.at[srcs.at[k + j]], gb,
                                          sg).wait()
                    scatters.append(
                        pltpu.async_copy(gb, acc_sh.at[dsts.at[k + j]], ss,
                                         add=True))
                for j, (gb, sg, ss) in enumerate(bufs):
                    scatters[j].wait()

                    @pl.when(k + 4 + j < SUP)
                    def _():
                        pltpu.async_copy(table.at[srcs.at[k + 4 + j]], gb, sg)

        @pl.loop(0, NSUP, step=2)
        def _(u):
            pltpu.make_async_copy(src_hbm.at[s, pl.ds(u * SUP, SUP)],
                                  src_a, sem_ia).wait()
            pltpu.make_async_copy(dst_hbm.at[s, pl.ds(u * SUP, SUP)],
                                  dst_a, sem_ia).wait()
            pltpu.async_copy(src_hbm.at[s, pl.ds((u + 1) * SUP, SUP)],
                             src_b, sem_ib)
            pltpu.async_copy(dst_hbm.at[s, pl.ds((u + 1) * SUP, SUP)],
                             dst_b, sem_ib)
            _process(src_a, dst_a)
            pltpu.make_async_copy(src_hbm.at[s, pl.ds((u + 1) * SUP, SUP)],
                                  src_b, sem_ib).wait()
            pltpu.make_async_copy(dst_hbm.at[s, pl.ds((u + 1) * SUP, SUP)],
                                  dst_b, sem_ib).wait()

            @pl.when(u + 2 < NSUP)
            def _():
                pltpu.async_copy(src_hbm.at[s, pl.ds((u + 2) * SUP, SUP)],
                                 src_a, sem_ia)
                pltpu.async_copy(dst_hbm.at[s, pl.ds((u + 2) * SUP, SUP)],
                                 dst_a, sem_ia)

            _process(src_b, dst_b)

        plsc.subcore_barrier()
        pltpu.sync_copy(acc_sh.at[pl.ds(s * RPS, RPS)],
                        out_hbm.at[c, pl.ds(s * RPS, RPS)])

    return agg_kernel(hws, src3, dst3)


def _bn(t, g, b):
    mu = jnp.mean(t, axis=0)
    var = jnp.mean((t - mu) ** 2, axis=0)
    return g * (t - mu) * lax.rsqrt(var + BN_EPS) + b


def _dis_col(dcol):
    return lax.rsqrt(dcol[0] + dcol[1] + 1.0)[:N]


def _dot(a, b):
    return jnp.dot(a, b, preferred_element_type=_f32)


def _enc_body(x_ref, we1, be1, g1, bt1, we2, be2, g2, bt2, we3, be3, wc1,
              u1_ref):
    h = _bn(_dot(x_ref[...], we1[...]) + be1[...], g1[...], bt1[...])
    h = jnp.maximum(h, 0.0)
    h = _bn(_dot(h, we2[...]) + be2[...], g2[...], bt2[...])
    h = jnp.maximum(h, 0.0)
    lat = _dot(h, we3[...]) + be3[...]
    u1_ref[...] = _dot(lat, wc1[...])


def _write_hws(hws_ref, us):
    hws_ref[0, :N, :] = us[:, :HALF]
    hws_ref[1, :N, :] = us[:, HALF:]
    z = jnp.zeros((NPAD - N, HALF), _f32)
    hws_ref[0, N:, :] = z
    hws_ref[1, N:, :] = z


def _scale_body(u_ref, dcol_ref, hws_ref):
    dis = _dis_col(dcol_ref[...])
    _write_hws(hws_ref, u_ref[...] * dis)


def _post_nores_body(agg_ref, dcol_ref, b_ref, g_ref, bb_ref, h_ref):
    dis = _dis_col(dcol_ref[...])
    t = jnp.concatenate([agg_ref[0, :N, :] * dis, agg_ref[1, :N, :] * dis],
                        axis=1) + b_ref[...]
    h_ref[...] = jnp.maximum(_bn(t, g_ref[...], bb_ref[...]), 0.0)


def _post_res_body(agg_ref, dcol_ref, b_ref, g_ref, bb_ref, res_ref, h_ref):
    dis = _dis_col(dcol_ref[...])
    t = jnp.concatenate([agg_ref[0, :N, :] * dis, agg_ref[1, :N, :] * dis],
                        axis=1) + b_ref[...]
    h_ref[...] = jnp.maximum(_bn(t, g_ref[...], bb_ref[...]) + res_ref[...],
                             0.0)


def _matscale_body(h_ref, dcol_ref, w_ref, hws_ref):
    dis = _dis_col(dcol_ref[...])
    _write_hws(hws_ref, _dot(h_ref[...], w_ref[...]) * dis)


def _head_body(agg_ref, dcol_ref, b_ref, g_ref, bb_ref, res_ref, wh_ref,
               bh_ref, out_ref):
    dis = _dis_col(dcol_ref[...])
    t = jnp.concatenate([agg_ref[0, :N, :] * dis, agg_ref[1, :N, :] * dis],
                        axis=1) + b_ref[...]
    h3 = jnp.maximum(_bn(t, g_ref[...], bb_ref[...]) + res_ref[...], 0.0)
    out_ref[...] = _dot(h3, wh_ref[...]) + bh_ref[...]


def _tc(body, out_shape, *args):
    return pl.pallas_call(
        body, out_shape=jax.ShapeDtypeStruct(out_shape, _f32))(*args)


def kernel(x, edge_index, We1, be1, g1, bt1, We2, be2, g2, bt2, We3, be3,
           Wc1, bc1, gn1, bnb1, Wc2, bc2, gn2, bnb2, Wc3, bc3, gn3, bnb3,
           Wh, bh):
    npad_e = EPAD - E
    src_p = jnp.concatenate(
        [edge_index[0], jnp.zeros((npad_e,), jnp.int32)]).reshape(
            NSUB, CPS, CHUNK)
    dst_p = jnp.concatenate(
        [edge_index[1],
         N + (jnp.arange(npad_e, dtype=jnp.int32) % (NPAD - N))]).reshape(
            NSUB, CPS, CHUNK)

    dst_w = dst_p.reshape(NCORE * NSUB, CPS // 2, CHUNK)
    deg2 = _sc_degree(dst_w)
    dcol = deg2.reshape(NCORE, NPAD, 1)

    u1 = _tc(_enc_body, (N, HID), x, We1, be1, g1, bt1, We2, be2, g2, bt2,
             We3, be3, Wc1)

    hws1 = _tc(_scale_body, (NCORE, NPAD, HALF), u1, dcol)
    agg1 = _sc_aggregate(hws1, src_p, dst_p)
    h1 = _tc(_post_nores_body, (N, HID), agg1, dcol, bc1, gn1, bnb1)

    hws2 = _tc(_matscale_body, (NCORE, NPAD, HALF), h1, dcol, Wc2)
    agg2 = _sc_aggregate(hws2, src_p, dst_p)
    h2 = _tc(_post_res_body, (N, HID), agg2, dcol, bc2, gn2, bnb2, h1)

    hws3 = _tc(_matscale_body, (NCORE, NPAD, HALF), h2, dcol, Wc3)
    agg3 = _sc_aggregate(hws3, src_p, dst_p)
    logits = _tc(_head_body, (N, NLAB), agg3, dcol, bc3, gn3, bnb3, h2,
                 Wh, bh)
    return logits
